# baseline (speedup 1.0000x reference)
import functools

import jax
import jax.numpy as jnp
from jax import lax
from jax.experimental import pallas as pl
from jax.experimental.pallas import tpu as pltpu
from jax.experimental.pallas import tpu_sc as plsc

D_IN = 1024
NL = 32768
NT = 32
K = 32
BL = 1024
NB = NL // BL
G = 128
NG = NL // G
NG_BLK = BL // G


def _enc_body(x_ref, w_ref, b_ref, pre_ref, gmax_ref):
    pre = lax.dot_general(
        x_ref[...], w_ref[...],
        dimension_numbers=(((1,), (1,)), ((), ())),
        preferred_element_type=jnp.float32,
    ) + b_ref[...]
    pre_ref[...] = pre
    cols = [jnp.max(pre[:, g * G:(g + 1) * G], axis=1, keepdims=True)
            for g in range(NG_BLK)]
    gmax_ref[0] = jnp.concatenate(cols, axis=1)


def _encode(x, w_enc, b_enc2d):
    return pl.pallas_call(
        _enc_body,
        grid=(NB,),
        in_specs=[
            pl.BlockSpec((NT, D_IN), lambda i: (0, 0)),
            pl.BlockSpec((BL, D_IN), lambda i: (i, 0)),
            pl.BlockSpec((1, BL), lambda i: (0, i)),
        ],
        out_specs=[
            pl.BlockSpec((NT, BL), lambda i: (0, i)),
            pl.BlockSpec((1, NT, NG_BLK), lambda i: (i, 0, 0)),
        ],
        out_shape=[
            jax.ShapeDtypeStruct((NT, NL), jnp.float32),
            jax.ShapeDtypeStruct((NB, NT, NG_BLK), jnp.float32),
        ],
        compiler_params=pltpu.CompilerParams(
            dimension_semantics=("arbitrary",),
        ),
    )(x, w_enc, b_enc2d)


NC = 2
NS = 16
L = 16
NEG = float("-inf")
BIGI = NL


def _scalar_tree(vec, combine):
    vals = [vec[i] for i in range(L)]
    while len(vals) > 1:
        vals = [combine(vals[i], vals[i + 1]) for i in range(0, len(vals) - 1, 2)] \
            + ([vals[-1]] if len(vals) % 2 else [])
    return vals[0]


def _merge_tree(pairs):
    while len(pairs) > 1:
        nxt = []
        for i in range(0, len(pairs) - 1, 2):
            am, ag = pairs[i]
            bm, bg = pairs[i + 1]
            takeb = (bm > am) | ((bm == am) & (bg < ag))
            nxt.append((jnp.where(takeb, bm, am), jnp.where(takeb, bg, ag)))
        if len(pairs) % 2:
            nxt.append(pairs[-1])
        pairs = nxt
    return pairs[0]


def _sc_body(pre_hbm, gmax_hbm, wdec_hbm, bdec_hbm, out_hbm,
             row_v, gmax_v, idx_v, rows_v, y_v, sem):
    wid = lax.axis_index("s") * NC + lax.axis_index("c")
    pltpu.sync_copy(pre_hbm.at[wid], row_v)
    pltpu.sync_copy(gmax_hbm.at[wid], gmax_v)
    pltpu.sync_copy(bdec_hbm, y_v)

    lanes = lax.iota(jnp.int32, L)

    def select_one(k, carry):
        i0, i1, v0, v1 = carry
        mv, gv = _merge_tree(
            [(gmax_v[pl.ds(c * L, L)], lanes + c * L)
             for c in range(NG // L)])
        m, g = _merge_tree([(mv[i], gv[i]) for i in range(L)])
        base = g * G
        ec = jnp.full((L,), BIGI, jnp.int32)
        for j in range(G // L):
            chunk = row_v[pl.ds(base + j * L, L)]
            ec = jnp.minimum(ec, jnp.where(chunk == m, lanes + (j * L), BIGI))
        e = base + _scalar_tree(ec, jnp.minimum)
        i0 = jnp.where(lanes == k, e, i0)
        i1 = jnp.where(lanes == k - L, e, i1)
        v0 = jnp.where(lanes == k, m, v0)
        v1 = jnp.where(lanes == k - L, m, v1)
        cid = (e // L) * L
        chunk = row_v[pl.ds(cid, L)]
        row_v[pl.ds(cid, L)] = jnp.where(lanes == e - cid, NEG, chunk)
        chunks = [row_v[pl.ds(base + j * L, L)] for j in range(G // L)]
        while len(chunks) > 1:
            chunks = [jnp.maximum(chunks[i], chunks[i + 1])
                      for i in range(0, len(chunks), 2)]
        nm = _scalar_tree(chunks[0], jnp.maximum)
        gcid = (g // L) * L
        gchunk = gmax_v[pl.ds(gcid, L)]
        gmax_v[pl.ds(gcid, L)] = jnp.where(lanes == g - gcid, nm, gchunk)
        return i0, i1, v0, v1

    zi = jnp.zeros((L,), jnp.int32)
    zf = jnp.zeros((L,), jnp.float32)
    i0, i1, v0, v1 = lax.fori_loop(0, K, select_one, (zi, zi, zf, zf))
    idx_v[pl.ds(0, L)] = i0
    idx_v[pl.ds(L, L)] = i1

    pltpu.async_copy(wdec_hbm.at[idx_v], rows_v, sem).wait()

    a = [jnp.full((L,), v0[k] if k < L else v1[k - L], jnp.float32)
         for k in range(K)]

    def acc(j, c):
        off = j * L
        parts = [y_v[pl.ds(off, L)]] + [None] * 3
        for k in range(K):
            p = k % 4
            term = a[k] * rows_v[k, pl.ds(off, L)]
            parts[p] = term if parts[p] is None else parts[p] + term
        y_v[pl.ds(off, L)] = (parts[0] + parts[1]) + (parts[2] + parts[3])
        return c

    lax.fori_loop(0, D_IN // L, acc, 0)

    pltpu.sync_copy(y_v, out_hbm.at[wid])


def _decode(pre, gmax, w_dec, b_dec):
    mesh = plsc.VectorSubcoreMesh(core_axis_name="c", subcore_axis_name="s",
                                  num_cores=NC, num_subcores=NS)
    f = pl.kernel(
        _sc_body,
        out_type=jax.ShapeDtypeStruct((NT, D_IN), jnp.float32),
        mesh=mesh,
        scratch_types=[
            pltpu.VMEM((NL,), jnp.float32),
            pltpu.VMEM((NG,), jnp.float32),
            pltpu.VMEM((K,), jnp.int32),
            pltpu.VMEM((K, D_IN), jnp.float32),
            pltpu.VMEM((D_IN,), jnp.float32),
            pltpu.SemaphoreType.DMA,
        ],
    )
    return f(pre, gmax, w_dec, b_dec)


def kernel(x, W_enc, b_enc, W_dec, b_dec):
    pre, gmax3d = _encode(x, W_enc, b_enc.reshape(1, NL))
    gmax = gmax3d.transpose(1, 0, 2).reshape(NT, NG)
    return _decode(pre, gmax, W_dec, b_dec)


# speedup vs baseline: 1.1285x; 1.1285x over previous
import functools

import jax
import jax.numpy as jnp
from jax import lax
from jax.experimental import pallas as pl
from jax.experimental.pallas import tpu as pltpu
from jax.experimental.pallas import tpu_sc as plsc

D_IN = 1024
NL = 32768
NT = 32
K = 32
BL = 2048
NB = NL // BL
G = 128
NG = NL // G
NG_BLK = BL // G


def _enc_body(x_ref, w_ref, b_ref, pre_ref, gmax_ref):
    pre = lax.dot_general(
        x_ref[...], w_ref[...],
        dimension_numbers=(((1,), (1,)), ((), ())),
        preferred_element_type=jnp.float32,
    ) + b_ref[...]
    pre_ref[...] = pre
    cols = [jnp.max(pre[:, g * G:(g + 1) * G], axis=1, keepdims=True)
            for g in range(NG_BLK)]
    gmax_ref[0] = jnp.concatenate(cols, axis=1)


def _encode(x, w_enc, b_enc2d):
    return pl.pallas_call(
        _enc_body,
        grid=(NB,),
        in_specs=[
            pl.BlockSpec((NT, D_IN), lambda i: (0, 0)),
            pl.BlockSpec((BL, D_IN), lambda i: (i, 0)),
            pl.BlockSpec((1, BL), lambda i: (0, i)),
        ],
        out_specs=[
            pl.BlockSpec((NT, BL), lambda i: (0, i)),
            pl.BlockSpec((1, NT, NG_BLK), lambda i: (i, 0, 0)),
        ],
        out_shape=[
            jax.ShapeDtypeStruct((NT, NL), jnp.float32),
            jax.ShapeDtypeStruct((NB, NT, NG_BLK), jnp.float32),
        ],
        compiler_params=pltpu.CompilerParams(
            dimension_semantics=("arbitrary",),
        ),
    )(x, w_enc, b_enc2d)


NC = 2
NS = 16
L = 16
NEG = float("-inf")
BIGI = NL


def _scalar_tree(vec, combine):
    vals = [vec[i] for i in range(L)]
    while len(vals) > 1:
        vals = [combine(vals[i], vals[i + 1]) for i in range(0, len(vals) - 1, 2)] \
            + ([vals[-1]] if len(vals) % 2 else [])
    return vals[0]


def _merge_tree(pairs):
    while len(pairs) > 1:
        nxt = []
        for i in range(0, len(pairs) - 1, 2):
            am, ag = pairs[i]
            bm, bg = pairs[i + 1]
            takeb = (bm > am) | ((bm == am) & (bg < ag))
            nxt.append((jnp.where(takeb, bm, am), jnp.where(takeb, bg, ag)))
        if len(pairs) % 2:
            nxt.append(pairs[-1])
        pairs = nxt
    return pairs[0]


def _sc_body(pre_hbm, gmax_hbm, wdec_hbm, bdec_hbm, out_hbm,
             row_v, gmax_v, idx_v, rows_v, y_v, sem):
    wid = lax.axis_index("s") * NC + lax.axis_index("c")
    pltpu.sync_copy(pre_hbm.at[wid], row_v)
    pltpu.sync_copy(gmax_hbm.at[:, wid], gmax_v)
    pltpu.sync_copy(bdec_hbm, y_v)

    lanes = lax.iota(jnp.int32, L)

    def select_one(k, carry):
        i0, i1, v0, v1 = carry
        mv, gv = _merge_tree(
            [(gmax_v[c, pl.ds(0, L)], lanes + c * L)
             for c in range(NG // L)])
        m, g = _merge_tree([(mv[i], gv[i]) for i in range(L)])
        base = g * G
        ec = jnp.full((L,), BIGI, jnp.int32)
        for j in range(G // L):
            chunk = row_v[pl.ds(base + j * L, L)]
            ec = jnp.minimum(ec, jnp.where(chunk == m, lanes + (j * L), BIGI))
        e = base + _scalar_tree(ec, jnp.minimum)
        i0 = jnp.where(lanes == k, e, i0)
        i1 = jnp.where(lanes == k - L, e, i1)
        v0 = jnp.where(lanes == k, m, v0)
        v1 = jnp.where(lanes == k - L, m, v1)
        cid = (e // L) * L
        chunk = row_v[pl.ds(cid, L)]
        row_v[pl.ds(cid, L)] = jnp.where(lanes == e - cid, NEG, chunk)
        chunks = [row_v[pl.ds(base + j * L, L)] for j in range(G // L)]
        while len(chunks) > 1:
            chunks = [jnp.maximum(chunks[i], chunks[i + 1])
                      for i in range(0, len(chunks), 2)]
        nm = _scalar_tree(chunks[0], jnp.maximum)
        gr = g // L
        gchunk = gmax_v[gr, pl.ds(0, L)]
        gmax_v[gr, pl.ds(0, L)] = jnp.where(lanes == g - gr * L, nm, gchunk)
        return i0, i1, v0, v1

    zi = jnp.zeros((L,), jnp.int32)
    zf = jnp.zeros((L,), jnp.float32)
    i0, i1, v0, v1 = lax.fori_loop(0, K, select_one, (zi, zi, zf, zf))
    idx_v[pl.ds(0, L)] = i0
    idx_v[pl.ds(L, L)] = i1

    pltpu.async_copy(wdec_hbm.at[idx_v], rows_v, sem).wait()

    a = [jnp.full((L,), v0[k] if k < L else v1[k - L], jnp.float32)
         for k in range(K)]

    def acc(j, c):
        off = j * L
        parts = [y_v[pl.ds(off, L)]] + [None] * 3
        for k in range(K):
            p = k % 4
            term = a[k] * rows_v[k, pl.ds(off, L)]
            parts[p] = term if parts[p] is None else parts[p] + term
        y_v[pl.ds(off, L)] = (parts[0] + parts[1]) + (parts[2] + parts[3])
        return c

    lax.fori_loop(0, D_IN // L, acc, 0)

    pltpu.sync_copy(y_v, out_hbm.at[wid])


def _decode(pre, gmax, w_dec, b_dec):
    mesh = plsc.VectorSubcoreMesh(core_axis_name="c", subcore_axis_name="s",
                                  num_cores=NC, num_subcores=NS)
    f = pl.kernel(
        _sc_body,
        out_type=jax.ShapeDtypeStruct((NT, D_IN), jnp.float32),
        mesh=mesh,
        scratch_types=[
            pltpu.VMEM((NL,), jnp.float32),
            pltpu.VMEM((NB, NG_BLK), jnp.float32),
            pltpu.VMEM((K,), jnp.int32),
            pltpu.VMEM((K, D_IN), jnp.float32),
            pltpu.VMEM((D_IN,), jnp.float32),
            pltpu.SemaphoreType.DMA,
        ],
    )
    return f(pre, gmax, w_dec, b_dec)


def kernel(x, W_enc, b_enc, W_dec, b_dec):
    pre, gmax3d = _encode(x, W_enc, b_enc.reshape(1, NL))
    return _decode(pre, gmax3d, W_dec, b_dec)


# speedup vs baseline: 1.1466x; 1.0161x over previous
import functools

import jax
import jax.numpy as jnp
from jax import lax
from jax.experimental import pallas as pl
from jax.experimental.pallas import tpu as pltpu
from jax.experimental.pallas import tpu_sc as plsc

D_IN = 1024
NL = 32768
NT = 32
K = 32
BL = 2048
NB = NL // BL
G = 128
NG = NL // G
NG_BLK = BL // G


def _enc_body(x_ref, w_ref, b_ref, pre_ref, gmax_ref):
    pre = lax.dot_general(
        x_ref[...], w_ref[...],
        dimension_numbers=(((1,), (1,)), ((), ())),
        preferred_element_type=jnp.float32,
    ) + b_ref[...]
    pre_ref[...] = pre
    cols = [jnp.max(pre[:, g * G:(g + 1) * G], axis=1, keepdims=True)
            for g in range(NG_BLK)]
    gmax_ref[0] = jnp.concatenate(cols, axis=1)


def _encode(x, w_enc, b_enc2d):
    return pl.pallas_call(
        _enc_body,
        grid=(NB,),
        in_specs=[
            pl.BlockSpec((NT, D_IN), lambda i: (0, 0)),
            pl.BlockSpec((BL, D_IN), lambda i: (i, 0)),
            pl.BlockSpec((1, BL), lambda i: (0, i)),
        ],
        out_specs=[
            pl.BlockSpec((NT, BL), lambda i: (0, i)),
            pl.BlockSpec((1, NT, NG_BLK), lambda i: (i, 0, 0)),
        ],
        out_shape=[
            jax.ShapeDtypeStruct((NT, NL), jnp.float32),
            jax.ShapeDtypeStruct((NB, NT, NG_BLK), jnp.float32),
        ],
        compiler_params=pltpu.CompilerParams(
            dimension_semantics=("arbitrary",),
        ),
    )(x, w_enc, b_enc2d)


NC = 2
NS = 16
L = 16
NEG = float("-inf")
BIGI = NL


def _scalar_tree(vec, combine):
    vals = [vec[i] for i in range(L)]
    while len(vals) > 1:
        vals = [combine(vals[i], vals[i + 1]) for i in range(0, len(vals) - 1, 2)] \
            + ([vals[-1]] if len(vals) % 2 else [])
    return vals[0]


def _merge_tree(pairs):
    while len(pairs) > 1:
        nxt = []
        for i in range(0, len(pairs) - 1, 2):
            am, ag = pairs[i]
            bm, bg = pairs[i + 1]
            takeb = (bm > am) | ((bm == am) & (bg < ag))
            nxt.append((jnp.where(takeb, bm, am), jnp.where(takeb, bg, ag)))
        if len(pairs) % 2:
            nxt.append(pairs[-1])
        pairs = nxt
    return pairs[0]


def _sc_body(pre_hbm, gmax_hbm, wdec_hbm, bdec_hbm, out_hbm,
             row_v, gmax_v, idx_v, rows_v, y_v, sem):
    wid = lax.axis_index("s") * NC + lax.axis_index("c")
    row_cp = pltpu.make_async_copy(pre_hbm.at[wid], row_v, sem)
    row_cp.start()
    pltpu.sync_copy(gmax_hbm.at[:, wid], gmax_v)
    pltpu.sync_copy(bdec_hbm, y_v)
    row_cp.wait()

    lanes = lax.iota(jnp.int32, L)

    def select_one(k, carry):
        i0, i1, v0, v1 = carry
        mv, gv = _merge_tree(
            [(gmax_v[c, pl.ds(0, L)], lanes + c * L)
             for c in range(NG // L)])
        m, g = _merge_tree([(mv[i], gv[i]) for i in range(L)])
        base = g * G
        ec = jnp.full((L,), BIGI, jnp.int32)
        m1 = jnp.full((L,), NEG, jnp.float32)
        m2 = jnp.full((L,), NEG, jnp.float32)
        for j in range(G // L):
            chunk = row_v[pl.ds(base + j * L, L)]
            ec = jnp.minimum(ec, jnp.where(chunk == m, lanes + (j * L), BIGI))
            m2 = jnp.maximum(m2, jnp.minimum(m1, chunk))
            m1 = jnp.maximum(m1, chunk)
        off = _scalar_tree(ec, jnp.minimum)
        e = base + off
        i0 = jnp.where(lanes == k, e, i0)
        i1 = jnp.where(lanes == k - L, e, i1)
        v0 = jnp.where(lanes == k, m, v0)
        v1 = jnp.where(lanes == k - L, m, v1)
        cid = (e // L) * L
        chunk = row_v[pl.ds(cid, L)]
        row_v[pl.ds(cid, L)] = jnp.where(lanes == e - cid, NEG, chunk)
        lane_e = off - (off // L) * L
        nm = _scalar_tree(jnp.where(lanes == lane_e, m2, m1), jnp.maximum)
        gr = g // L
        gchunk = gmax_v[gr, pl.ds(0, L)]
        gmax_v[gr, pl.ds(0, L)] = jnp.where(lanes == g - gr * L, nm, gchunk)
        return i0, i1, v0, v1

    zi = jnp.zeros((L,), jnp.int32)
    zf = jnp.zeros((L,), jnp.float32)
    i0, i1, v0, v1 = lax.fori_loop(0, K, select_one, (zi, zi, zf, zf))
    idx_v[pl.ds(0, L)] = i0
    idx_v[pl.ds(L, L)] = i1

    gather_cp = pltpu.make_async_copy(wdec_hbm.at[idx_v], rows_v, sem)
    gather_cp.start()
    a = [jnp.full((L,), v0[k] if k < L else v1[k - L], jnp.float32)
         for k in range(K)]
    gather_cp.wait()

    def acc(j, c):
        off = j * L
        parts = [y_v[pl.ds(off, L)]] + [None] * 3
        for k in range(K):
            p = k % 4
            term = a[k] * rows_v[k, pl.ds(off, L)]
            parts[p] = term if parts[p] is None else parts[p] + term
        y_v[pl.ds(off, L)] = (parts[0] + parts[1]) + (parts[2] + parts[3])
        return c

    lax.fori_loop(0, D_IN // L, acc, 0)

    pltpu.sync_copy(y_v, out_hbm.at[wid])


def _decode(pre, gmax, w_dec, b_dec):
    mesh = plsc.VectorSubcoreMesh(core_axis_name="c", subcore_axis_name="s",
                                  num_cores=NC, num_subcores=NS)
    f = pl.kernel(
        _sc_body,
        out_type=jax.ShapeDtypeStruct((NT, D_IN), jnp.float32),
        mesh=mesh,
        scratch_types=[
            pltpu.VMEM((NL,), jnp.float32),
            pltpu.VMEM((NB, NG_BLK), jnp.float32),
            pltpu.VMEM((K,), jnp.int32),
            pltpu.VMEM((K, D_IN), jnp.float32),
            pltpu.VMEM((D_IN,), jnp.float32),
            pltpu.SemaphoreType.DMA,
        ],
    )
    return f(pre, gmax, w_dec, b_dec)


def kernel(x, W_enc, b_enc, W_dec, b_dec):
    pre, gmax3d = _encode(x, W_enc, b_enc.reshape(1, NL))
    return _decode(pre, gmax3d, W_dec, b_dec)


# speedup vs baseline: 1.1659x; 1.0168x over previous
import functools

import jax
import jax.numpy as jnp
from jax import lax
from jax.experimental import pallas as pl
from jax.experimental.pallas import tpu as pltpu
from jax.experimental.pallas import tpu_sc as plsc

D_IN = 1024
NL = 32768
NT = 32
K = 32
BL = 2048
NB = NL // BL
G = 128
NG = NL // G
NG_BLK = BL // G


def _enc_body(x_ref, w_ref, b_ref, pre_ref, gmax_ref):
    pre = lax.dot_general(
        x_ref[...], w_ref[...],
        dimension_numbers=(((1,), (1,)), ((), ())),
        preferred_element_type=jnp.float32,
    ) + b_ref[...]
    pre_ref[...] = pre
    cols = [jnp.max(pre[:, g * G:(g + 1) * G], axis=1, keepdims=True)
            for g in range(NG_BLK)]
    gmax_ref[0] = jnp.concatenate(cols, axis=1)


def _encode(x, w_enc, b_enc2d):
    return pl.pallas_call(
        _enc_body,
        grid=(NB,),
        in_specs=[
            pl.BlockSpec((NT, D_IN), lambda i: (0, 0)),
            pl.BlockSpec((BL, D_IN), lambda i: (i, 0)),
            pl.BlockSpec((1, BL), lambda i: (0, i)),
        ],
        out_specs=[
            pl.BlockSpec((NT, BL), lambda i: (0, i)),
            pl.BlockSpec((1, NT, NG_BLK), lambda i: (i, 0, 0)),
        ],
        out_shape=[
            jax.ShapeDtypeStruct((NT, NL), jnp.float32),
            jax.ShapeDtypeStruct((NB, NT, NG_BLK), jnp.float32),
        ],
        compiler_params=pltpu.CompilerParams(
            dimension_semantics=("arbitrary",),
        ),
    )(x, w_enc, b_enc2d)


NC = 2
NS = 16
L = 16
NEG = float("-inf")
BIGI = NL


def _scalar_tree(vec, combine):
    vals = [vec[i] for i in range(L)]
    while len(vals) > 1:
        vals = [combine(vals[i], vals[i + 1]) for i in range(0, len(vals) - 1, 2)] \
            + ([vals[-1]] if len(vals) % 2 else [])
    return vals[0]


def _merge_tree(pairs):
    while len(pairs) > 1:
        nxt = []
        for i in range(0, len(pairs) - 1, 2):
            am, ag = pairs[i]
            bm, bg = pairs[i + 1]
            takeb = (bm > am) | ((bm == am) & (bg < ag))
            nxt.append((jnp.where(takeb, bm, am), jnp.where(takeb, bg, ag)))
        if len(pairs) % 2:
            nxt.append(pairs[-1])
        pairs = nxt
    return pairs[0]


def _sc_body(pre_hbm, gmax_hbm, wdec_hbm, bdec_hbm, out_hbm,
             row_v, gmax_v, rows_v, y_v, sem, gsem):
    wid = lax.axis_index("s") * NC + lax.axis_index("c")
    row_cp = pltpu.make_async_copy(pre_hbm.at[wid], row_v, sem)
    row_cp.start()
    pltpu.sync_copy(gmax_hbm.at[:, wid], gmax_v)
    pltpu.sync_copy(bdec_hbm, y_v)
    row_cp.wait()

    lanes = lax.iota(jnp.int32, L)

    def select_one(k, carry):
        i0, i1, v0, v1 = carry
        mv, gv = _merge_tree(
            [(gmax_v[c, pl.ds(0, L)], lanes + c * L)
             for c in range(NG // L)])
        m, g = _merge_tree([(mv[i], gv[i]) for i in range(L)])
        base = g * G
        ec = jnp.full((L,), BIGI, jnp.int32)
        m1 = jnp.full((L,), NEG, jnp.float32)
        m2 = jnp.full((L,), NEG, jnp.float32)
        for j in range(G // L):
            chunk = row_v[pl.ds(base + j * L, L)]
            ec = jnp.minimum(ec, jnp.where(chunk == m, lanes + (j * L), BIGI))
            m2 = jnp.maximum(m2, jnp.minimum(m1, chunk))
            m1 = jnp.maximum(m1, chunk)
        off = _scalar_tree(ec, jnp.minimum)
        e = base + off
        i0 = jnp.where(lanes == k, e, i0)
        i1 = jnp.where(lanes == k - L, e, i1)
        v0 = jnp.where(lanes == k, m, v0)
        v1 = jnp.where(lanes == k - L, m, v1)
        pltpu.make_async_copy(wdec_hbm.at[e], rows_v.at[k], gsem).start()
        cid = (e // L) * L
        chunk = row_v[pl.ds(cid, L)]
        row_v[pl.ds(cid, L)] = jnp.where(lanes == e - cid, NEG, chunk)
        lane_e = off - (off // L) * L
        nm = _scalar_tree(jnp.where(lanes == lane_e, m2, m1), jnp.maximum)
        gr = g // L
        gchunk = gmax_v[gr, pl.ds(0, L)]
        gmax_v[gr, pl.ds(0, L)] = jnp.where(lanes == g - gr * L, nm, gchunk)
        return i0, i1, v0, v1

    zi = jnp.zeros((L,), jnp.int32)
    zf = jnp.zeros((L,), jnp.float32)
    i0, i1, v0, v1 = lax.fori_loop(0, K, select_one, (zi, zi, zf, zf))

    a = [jnp.full((L,), v0[k] if k < L else v1[k - L], jnp.float32)
         for k in range(K)]
    pltpu.make_async_copy(wdec_hbm.at[pl.ds(0, K)], rows_v, gsem).wait()

    def acc(j, c):
        off = j * L
        parts = [y_v[pl.ds(off, L)]] + [None] * 3
        for k in range(K):
            p = k % 4
            term = a[k] * rows_v[k, pl.ds(off, L)]
            parts[p] = term if parts[p] is None else parts[p] + term
        y_v[pl.ds(off, L)] = (parts[0] + parts[1]) + (parts[2] + parts[3])
        return c

    lax.fori_loop(0, D_IN // L, acc, 0)

    pltpu.sync_copy(y_v, out_hbm.at[wid])


def _decode(pre, gmax, w_dec, b_dec):
    mesh = plsc.VectorSubcoreMesh(core_axis_name="c", subcore_axis_name="s",
                                  num_cores=NC, num_subcores=NS)
    f = pl.kernel(
        _sc_body,
        out_type=jax.ShapeDtypeStruct((NT, D_IN), jnp.float32),
        mesh=mesh,
        scratch_types=[
            pltpu.VMEM((NL,), jnp.float32),
            pltpu.VMEM((NB, NG_BLK), jnp.float32),
            pltpu.VMEM((K, D_IN), jnp.float32),
            pltpu.VMEM((D_IN,), jnp.float32),
            pltpu.SemaphoreType.DMA,
            pltpu.SemaphoreType.DMA,
        ],
    )
    return f(pre, gmax, w_dec, b_dec)


def kernel(x, W_enc, b_enc, W_dec, b_dec):
    pre, gmax3d = _encode(x, W_enc, b_enc.reshape(1, NL))
    return _decode(pre, gmax3d, W_dec, b_dec)


# speedup vs baseline: 1.1675x; 1.0014x over previous
import jax
import jax.numpy as jnp
from jax import lax
from jax.experimental import pallas as pl
from jax.experimental.pallas import tpu as pltpu
from jax.experimental.pallas import tpu_sc as plsc

D_IN = 1024
NL = 32768
NT = 32
K = 32
BL = 2048
NB = NL // BL
G = 128
NG = NL // G
NG_BLK = BL // G


def _enc_body(x_ref, w_ref, b_ref, pre_ref, gmax_ref):
    pre = lax.dot_general(
        x_ref[...], w_ref[...],
        dimension_numbers=(((1,), (1,)), ((), ())),
        preferred_element_type=jnp.float32,
    ) + b_ref[...]
    pre_ref[...] = pre
    cols = [jnp.max(pre[:, g * G:(g + 1) * G], axis=1, keepdims=True)
            for g in range(NG_BLK)]
    gmax_ref[0] = jnp.concatenate(cols, axis=1)


def _encode(x, w_enc, b_enc2d):
    return pl.pallas_call(
        _enc_body,
        grid=(NB,),
        in_specs=[
            pl.BlockSpec((NT, D_IN), lambda i: (0, 0)),
            pl.BlockSpec((BL, D_IN), lambda i: (i, 0)),
            pl.BlockSpec((1, BL), lambda i: (0, i)),
        ],
        out_specs=[
            pl.BlockSpec((NT, BL), lambda i: (0, i)),
            pl.BlockSpec((1, NT, NG_BLK), lambda i: (i, 0, 0)),
        ],
        out_shape=[
            jax.ShapeDtypeStruct((NT, NL), jnp.float32),
            jax.ShapeDtypeStruct((NB, NT, NG_BLK), jnp.float32),
        ],
        compiler_params=pltpu.CompilerParams(
            dimension_semantics=("arbitrary",),
        ),
    )(x, w_enc, b_enc2d)


NC = 2
NS = 16
L = 16
NEG = float("-inf")
BIGI = NL


def _scalar_tree(vec, combine):
    vals = [vec[i] for i in range(L)]
    while len(vals) > 1:
        vals = [combine(vals[i], vals[i + 1]) for i in range(0, len(vals) - 1, 2)] \
            + ([vals[-1]] if len(vals) % 2 else [])
    return vals[0]


def _merge_tree(pairs):
    while len(pairs) > 1:
        nxt = []
        for i in range(0, len(pairs) - 1, 2):
            am, ag = pairs[i]
            bm, bg = pairs[i + 1]
            takeb = (bm > am) | ((bm == am) & (bg < ag))
            nxt.append((jnp.where(takeb, bm, am), jnp.where(takeb, bg, ag)))
        if len(pairs) % 2:
            nxt.append(pairs[-1])
        pairs = nxt
    return pairs[0]


def _sc_body(pre_hbm, gmax_hbm, wdec_hbm, bdec_hbm, out_hbm,
             row_v, gmax_v, rows_v, y_v, sem, gsem):
    wid = lax.axis_index("s") * NC + lax.axis_index("c")
    row_cp = pltpu.make_async_copy(pre_hbm.at[wid], row_v, sem)
    row_cp.start()
    pltpu.sync_copy(gmax_hbm.at[:, wid], gmax_v)
    pltpu.sync_copy(bdec_hbm, y_v)
    row_cp.wait()

    lanes = lax.iota(jnp.int32, L)

    def select_one(k, carry):
        i0, i1, v0, v1 = carry
        mv, gv = _merge_tree(
            [(gmax_v[c, pl.ds(0, L)], lanes + c * L)
             for c in range(NG // L)])
        m, g = _merge_tree([(mv[i], gv[i]) for i in range(L)])
        base = g * G
        ec = jnp.full((L,), BIGI, jnp.int32)
        m1 = jnp.full((L,), NEG, jnp.float32)
        m2 = jnp.full((L,), NEG, jnp.float32)
        for j in range(G // L):
            chunk = row_v[pl.ds(base + j * L, L)]
            ec = jnp.minimum(ec, jnp.where(chunk == m, lanes + (j * L), BIGI))
            m2 = jnp.maximum(m2, jnp.minimum(m1, chunk))
            m1 = jnp.maximum(m1, chunk)
        off = _scalar_tree(ec, jnp.minimum)
        e = base + off
        i0 = jnp.where(lanes == k, e, i0)
        i1 = jnp.where(lanes == k - L, e, i1)
        v0 = jnp.where(lanes == k, m, v0)
        v1 = jnp.where(lanes == k - L, m, v1)
        pltpu.make_async_copy(wdec_hbm.at[e], rows_v.at[k], gsem).start()
        cid = (e // L) * L
        chunk = row_v[pl.ds(cid, L)]
        row_v[pl.ds(cid, L)] = jnp.where(lanes == e - cid, NEG, chunk)
        lane_e = off - (off // L) * L
        nm = _scalar_tree(jnp.where(lanes == lane_e, m2, m1), jnp.maximum)
        gr = g // L
        gchunk = gmax_v[gr, pl.ds(0, L)]
        gmax_v[gr, pl.ds(0, L)] = jnp.where(lanes == g - gr * L, nm, gchunk)
        return i0, i1, v0, v1

    zi = jnp.zeros((L,), jnp.int32)
    zf = jnp.zeros((L,), jnp.float32)
    i0, i1, v0, v1 = lax.fori_loop(0, K, select_one, (zi, zi, zf, zf))

    a = [jnp.full((L,), v0[k] if k < L else v1[k - L], jnp.float32)
         for k in range(K)]
    pltpu.make_async_copy(wdec_hbm.at[pl.ds(0, K)], rows_v, gsem).wait()

    def acc(j, c):
        off = j * L
        parts = [y_v[pl.ds(off, L)]] + [None] * 3
        for k in range(K):
            p = k % 4
            term = a[k] * rows_v[k, pl.ds(off, L)]
            parts[p] = term if parts[p] is None else parts[p] + term
        y_v[pl.ds(off, L)] = (parts[0] + parts[1]) + (parts[2] + parts[3])
        return c

    lax.fori_loop(0, D_IN // L, acc, 0)

    pltpu.sync_copy(y_v, out_hbm.at[wid])


def _decode(pre, gmax, w_dec, b_dec):
    mesh = plsc.VectorSubcoreMesh(core_axis_name="c", subcore_axis_name="s",
                                  num_cores=NC, num_subcores=NS)
    f = pl.kernel(
        _sc_body,
        out_type=jax.ShapeDtypeStruct((NT, D_IN), jnp.float32),
        mesh=mesh,
        scratch_types=[
            pltpu.VMEM((NL,), jnp.float32),
            pltpu.VMEM((NB, NG_BLK), jnp.float32),
            pltpu.VMEM((K, D_IN), jnp.float32),
            pltpu.VMEM((D_IN,), jnp.float32),
            pltpu.SemaphoreType.DMA,
            pltpu.SemaphoreType.DMA,
        ],
    )
    return f(pre, gmax, w_dec, b_dec)


def kernel(x, W_enc, b_enc, W_dec, b_dec):
    pre, gmax3d = _encode(x, W_enc, b_enc.reshape(1, NL))
    return _decode(pre, gmax3d, W_dec, b_dec)
